# Initial kernel scaffold; baseline (speedup 1.0000x reference)
import jax
import jax.numpy as jnp
from jax.experimental import pallas as pl


def kernel(prediction, bin_values, theta):
    raise NotImplementedError("write your pallas kernel here")


# speedup vs baseline: 1.0938x; 1.0938x over previous
import jax
import jax.numpy as jnp
from jax import lax
from jax.experimental import pallas as pl
from jax.experimental.pallas import tpu as pltpu
from jax.experimental.pallas import tpu_sc as plsc

_L = 16
_NBINS = 21


def _gather(vec, idx):
    return vec.at[idx].get(mode="promise_in_bounds")


def _sc_body(pred_hbm, bins_hbm, theta_hbm, out_hbm, pred_v, bins_v, theta_v, out_v):
    cid = lax.axis_index("c")
    sid = lax.axis_index("s")

    @pl.when(jnp.logical_and(cid == 0, sid == 0))
    def _():
        pltpu.sync_copy(pred_hbm, pred_v.at[pl.ds(0, 1)])
        pltpu.sync_copy(bins_hbm, bins_v.at[pl.ds(_L, _NBINS)])
        pltpu.sync_copy(theta_hbm, theta_v.at[pl.ds(0, _NBINS)])

        lane = lax.iota(jnp.int32, _L)
        zero = jnp.zeros((_L,), jnp.float32)
        inf = jnp.full((_L,), jnp.inf, jnp.float32)
        pred = _gather(pred_v[pl.ds(0, _L)], jnp.zeros((_L,), jnp.int32))

        r = zero
        for half in range(2):
            base = _L + half * _L
            cur = bins_v[pl.ds(base, _L)]
            prev = bins_v[pl.ds(base - 1, _L)]
            theta = theta_v[pl.ds(half * _L, _L)]
            if half == 0:
                upper = cur
                lower = jnp.where(lane == 0, -inf, prev)
                live = lane == lane
            else:
                upper = jnp.where(lane == _NBINS - 1 - _L, inf, cur)
                lower = prev
                live = lane <= _NBINS - 1 - _L
            mask = jnp.logical_and(
                jnp.logical_and(pred <= upper, lower < pred), live
            )
            r = r + jnp.where(mask, cur + theta, zero)
        for k in (1, 2, 4, 8):
            r = r + _gather(r, lane ^ k)
        out_v[...] = r
        pltpu.sync_copy(out_v.at[pl.ds(0, 1)], out_hbm)


kernel = jax.jit(
    pl.kernel(
        _sc_body,
        out_type=jax.ShapeDtypeStruct((1,), jnp.float32),
        mesh=plsc.VectorSubcoreMesh(
            core_axis_name="c", subcore_axis_name="s", num_cores=2, num_subcores=16
        ),
        scratch_types=[
            pltpu.VMEM((_L,), jnp.float32),
            pltpu.VMEM((3 * _L,), jnp.float32),
            pltpu.VMEM((2 * _L,), jnp.float32),
            pltpu.VMEM((_L,), jnp.float32),
        ],
    )
)


# speedup vs baseline: 1.1262x; 1.0297x over previous
import jax
import jax.numpy as jnp
from jax import lax
from jax.experimental import pallas as pl
from jax.experimental.pallas import tpu as pltpu
from jax.experimental.pallas import tpu_sc as plsc

_L = 16
_NBINS = 21
_PACKED = 9 * _L


def _gather(vec, idx):
    return vec.at[idx].get(mode="promise_in_bounds")


def _sc_body(packed_hbm, out_hbm, buf_v, out_v):
    cid = lax.axis_index("c")
    sid = lax.axis_index("s")

    @pl.when(jnp.logical_and(cid == 0, sid == 0))
    def _():
        pltpu.sync_copy(packed_hbm, buf_v)
        pred = buf_v[pl.ds(0, _L)]
        zero = jnp.zeros((_L,), jnp.float32)
        r = zero
        for half in range(2):
            upper = buf_v[pl.ds(_L + half * _L, _L)]
            lower = buf_v[pl.ds(3 * _L + half * _L, _L)]
            val = buf_v[pl.ds(5 * _L + half * _L, _L)] + buf_v[
                pl.ds(7 * _L + half * _L, _L)
            ]
            mask = jnp.logical_and(pred <= upper, lower < pred)
            r = r + jnp.where(mask, val, zero)
        lane = lax.iota(jnp.int32, _L)
        for k in (1, 2, 4, 8):
            r = r + _gather(r, lane ^ k)
        out_v[...] = r
        pltpu.sync_copy(out_v.at[pl.ds(0, 1)], out_hbm)


_sc_call = pl.kernel(
    _sc_body,
    out_type=jax.ShapeDtypeStruct((1,), jnp.float32),
    mesh=plsc.VectorSubcoreMesh(
        core_axis_name="c", subcore_axis_name="s", num_cores=2, num_subcores=16
    ),
    scratch_types=[
        pltpu.VMEM((_PACKED,), jnp.float32),
        pltpu.VMEM((_L,), jnp.float32),
    ],
)


@jax.jit
def kernel(prediction, bin_values, theta):
    pad = 2 * _L - _NBINS
    inf = jnp.full((pad,), jnp.inf, jnp.float32)
    zpad = jnp.zeros((pad,), jnp.float32)
    ninf1 = jnp.full((1,), -jnp.inf, jnp.float32)
    upper = jnp.concatenate(
        [bin_values[: _NBINS - 1], jnp.full((1,), jnp.inf, jnp.float32), inf]
    )
    lower = jnp.concatenate([ninf1, bin_values[: _NBINS - 1], inf])
    packed = jnp.concatenate(
        [
            jnp.broadcast_to(prediction, (_L,)),
            upper,
            lower,
            jnp.concatenate([bin_values, zpad]),
            jnp.concatenate([theta, zpad]),
        ]
    )
    return _sc_call(packed)


# speedup vs baseline: 1.2313x; 1.0933x over previous
import jax
import jax.numpy as jnp
from jax import lax
from jax.experimental import pallas as pl
from jax.experimental.pallas import tpu as pltpu
from jax.experimental.pallas import tpu_sc as plsc

_NBINS = 21
_PACKED = 48


def _scs_body(packed_hbm, out_hbm, buf_s, out_s):
    cid = lax.axis_index("c")

    @pl.when(cid == 0)
    def _():
        pltpu.sync_copy(packed_hbm, buf_s)
        pred = buf_s[0]
        res = buf_s[1 + _NBINS - 1] + buf_s[1 + _NBINS + _NBINS - 1]
        for j in range(_NBINS - 2, -1, -1):
            hit = pred <= buf_s[1 + j]
            res = jnp.where(hit, buf_s[1 + j] + buf_s[1 + _NBINS + j], res)
        out_s[0] = res
        pltpu.sync_copy(out_s, out_hbm)


_scs_call = pl.kernel(
    _scs_body,
    out_type=jax.ShapeDtypeStruct((1,), jnp.float32),
    mesh=plsc.ScalarSubcoreMesh(axis_name="c", num_cores=2),
    scratch_types=[
        pltpu.SMEM((_PACKED,), jnp.float32),
        pltpu.SMEM((1,), jnp.float32),
    ],
)


@jax.jit
def kernel(prediction, bin_values, theta):
    packed = jnp.concatenate(
        [
            prediction,
            bin_values,
            theta,
            jnp.zeros((_PACKED - 1 - 2 * _NBINS,), jnp.float32),
        ]
    )
    return _scs_call(packed)


# speedup vs baseline: 1.3358x; 1.0849x over previous
import jax
import jax.numpy as jnp
from jax import lax
from jax.experimental import pallas as pl
from jax.experimental.pallas import tpu as pltpu
from jax.experimental.pallas import tpu_sc as plsc

_NBINS = 21
_PACKED = 48


def _scs_body(packed_hbm, out_hbm, buf_s, out_s):
    cid = lax.axis_index("c")

    @pl.when(cid == 0)
    def _():
        pltpu.sync_copy(packed_hbm, buf_s)
        pred = buf_s[0]
        res = buf_s[1 + _NBINS - 1] + buf_s[1 + _NBINS + _NBINS - 1]
        for j in range(_NBINS - 2, -1, -1):
            hit = pred <= buf_s[1 + j]
            res = jnp.where(hit, buf_s[1 + j] + buf_s[1 + _NBINS + j], res)
        out_s[0] = res
        pltpu.sync_copy(out_s, out_hbm)


_scs_call = pl.kernel(
    _scs_body,
    out_type=jax.ShapeDtypeStruct((1,), jnp.float32),
    mesh=plsc.ScalarSubcoreMesh(axis_name="c", num_cores=1),
    scratch_types=[
        pltpu.SMEM((_PACKED,), jnp.float32),
        pltpu.SMEM((1,), jnp.float32),
    ],
)


@jax.jit
def kernel(prediction, bin_values, theta):
    packed = jnp.concatenate(
        [
            prediction,
            bin_values,
            theta,
            jnp.zeros((_PACKED - 1 - 2 * _NBINS,), jnp.float32),
        ]
    )
    return _scs_call(packed)
